# baseline (speedup 1.0000x reference)
import math

import jax
import jax.numpy as jnp
from jax.experimental import pallas as pl
from jax.experimental.pallas import tpu as pltpu

HIDDEN = 256
IMAGE_SIZE = 1008
GRID = 72
B = 32
NPTS = 65
NSEM = 8
TWO_PI = 2.0 * math.pi

PIX = GRID * GRID
ROWS = HIDDEN * PIX // 128
SLAB = 2 * PIX // 128


def _body(px_ref, py_ref, lab_ref, pos_ref, tab_ref, nap_ref,
          ev_ref, od_ref, sparse_ref, dense_ref, pat_ref, sem):
    rr = jax.lax.broadcasted_iota(jnp.int32, (ROWS, 1), 0) % SLAB
    lane = jax.lax.broadcasted_iota(jnp.int32, (ROWS, 128), 1)
    use_odd = rr * 128 + lane >= PIX
    pat_ref[...] = jnp.where(use_odd, od_ref[...], ev_ref[...])

    copies = [
        pltpu.make_async_copy(pat_ref,
                              dense_ref.at[pl.ds(b * ROWS, ROWS)],
                              sem.at[b % NSEM])
        for b in range(B)
    ]
    for c in copies:
        c.start()

    px = px_ref[...]
    py = py_ref[...]
    labels = lab_ref[...]
    inv = 1.0 / IMAGE_SIZE
    bf = jnp.bfloat16
    cx = (2.0 * (px * inv) - 1.0).astype(bf).astype(jnp.float32)
    cy = (2.0 * (py * inv) - 1.0).astype(bf).astype(jnp.float32)
    p0 = pos_ref[0:1, :].reshape(1, 1, HIDDEN // 2)
    p1 = pos_ref[1:2, :].reshape(1, 1, HIDDEN // 2)
    p0 = p0.astype(bf).astype(jnp.float32)
    p1 = p1.astype(bf).astype(jnp.float32)
    c = TWO_PI * (cx * p0 + cy * p1)
    pe = jnp.concatenate([jnp.sin(c), jnp.cos(c)], axis=-1)
    nap = nap_ref[...].reshape(1, 1, HIDDEN)
    pe = jnp.where(labels == -1, nap, pe)
    pe = pe * (labels != -10).astype(pe.dtype)
    lc = jnp.maximum(labels, 0)
    e0 = tab_ref[0:1, :].reshape(1, 1, HIDDEN)
    e1 = tab_ref[1:2, :].reshape(1, 1, HIDDEN)
    e2 = tab_ref[2:3, :].reshape(1, 1, HIDDEN)
    e3 = tab_ref[3:4, :].reshape(1, 1, HIDDEN)
    pemb = jnp.where(lc == 0, e0,
                     jnp.where(lc == 1, e1,
                               jnp.where(lc == 2, e2, e3)))
    is_pos = (labels >= 0).astype(pe.dtype)
    sparse_ref[...] = pe + pemb * is_pos

    for c in copies:
        c.wait()


def kernel(input_points, input_labels, positional_embedding, point_embed,
           not_a_point_embed, no_mask_embed):
    pts = input_points + 0.5
    pts = jnp.concatenate([pts, jnp.zeros((B, 1, 2), pts.dtype)], axis=1)
    px = pts[..., 0:1]
    py = pts[..., 1:2]
    labels = jnp.concatenate(
        [input_labels, -jnp.ones((B, 1), input_labels.dtype)],
        axis=1)[..., None]
    nm = no_mask_embed.reshape(HIDDEN)
    ev = jnp.repeat(nm[0::2], SLAB)[:, None]
    od = jnp.repeat(nm[1::2], SLAB)[:, None]

    sparse, dense_flat = pl.pallas_call(
        _body,
        in_specs=[
            pl.BlockSpec((B, NPTS, 1), lambda: (0, 0, 0)),
            pl.BlockSpec((B, NPTS, 1), lambda: (0, 0, 0)),
            pl.BlockSpec((B, NPTS, 1), lambda: (0, 0, 0)),
            pl.BlockSpec((2, HIDDEN // 2), lambda: (0, 0)),
            pl.BlockSpec((4, HIDDEN), lambda: (0, 0)),
            pl.BlockSpec((1, HIDDEN), lambda: (0, 0)),
            pl.BlockSpec((ROWS, 1), lambda: (0, 0)),
            pl.BlockSpec((ROWS, 1), lambda: (0, 0)),
        ],
        out_specs=[
            pl.BlockSpec((B, NPTS, HIDDEN), lambda: (0, 0, 0)),
            pl.BlockSpec(memory_space=pltpu.MemorySpace.HBM),
        ],
        out_shape=[
            jax.ShapeDtypeStruct((B, NPTS, HIDDEN), jnp.float32),
            jax.ShapeDtypeStruct((B * ROWS, 128), jnp.float32),
        ],
        scratch_shapes=[
            pltpu.MemorySpace.VMEM((ROWS, 128), jnp.float32),
            pltpu.SemaphoreType.DMA((NSEM,)),
        ],
    )(px, py, labels, positional_embedding, point_embed,
      not_a_point_embed, ev, od)
    dense = dense_flat.reshape(B, HIDDEN, GRID, GRID)
    return sparse, dense


# speedup vs baseline: 8.9975x; 8.9975x over previous
import math

import jax
import jax.numpy as jnp
from jax.experimental import pallas as pl
from jax.experimental.pallas import tpu as pltpu

HIDDEN = 256
IMAGE_SIZE = 1008
GRID = 72
B = 32
NPTS = 65
NSEM = 8
TWO_PI = 2.0 * math.pi


def _body(px_ref, py_ref, lab_ref, pos_ref, tab_ref, nap_ref, nm_ref,
          sparse_ref, dense_ref, pat_ref, sem):
    pat_ref[...] = jnp.broadcast_to(nm_ref[...].reshape(1, 1, HIDDEN),
                                    (GRID, GRID, HIDDEN))

    copies = [
        pltpu.make_async_copy(pat_ref, dense_ref.at[b], sem.at[b % NSEM])
        for b in range(B)
    ]
    for c in copies:
        c.start()

    px = px_ref[...]
    py = py_ref[...]
    labels = lab_ref[...]
    inv = 1.0 / IMAGE_SIZE
    bf = jnp.bfloat16
    cx = (2.0 * (px * inv) - 1.0).astype(bf).astype(jnp.float32)
    cy = (2.0 * (py * inv) - 1.0).astype(bf).astype(jnp.float32)
    p0 = pos_ref[0:1, :].reshape(1, 1, HIDDEN // 2)
    p1 = pos_ref[1:2, :].reshape(1, 1, HIDDEN // 2)
    p0 = p0.astype(bf).astype(jnp.float32)
    p1 = p1.astype(bf).astype(jnp.float32)
    c = TWO_PI * (cx * p0 + cy * p1)
    pe = jnp.concatenate([jnp.sin(c), jnp.cos(c)], axis=-1)
    nap = nap_ref[...].reshape(1, 1, HIDDEN)
    pe = jnp.where(labels == -1, nap, pe)
    pe = pe * (labels != -10).astype(pe.dtype)
    lc = jnp.maximum(labels, 0)
    e0 = tab_ref[0:1, :].reshape(1, 1, HIDDEN)
    e1 = tab_ref[1:2, :].reshape(1, 1, HIDDEN)
    e2 = tab_ref[2:3, :].reshape(1, 1, HIDDEN)
    e3 = tab_ref[3:4, :].reshape(1, 1, HIDDEN)
    pemb = jnp.where(lc == 0, e0,
                     jnp.where(lc == 1, e1,
                               jnp.where(lc == 2, e2, e3)))
    is_pos = (labels >= 0).astype(pe.dtype)
    sparse_ref[...] = pe + pemb * is_pos

    for c in copies:
        c.wait()


def kernel(input_points, input_labels, positional_embedding, point_embed,
           not_a_point_embed, no_mask_embed):
    pts = input_points + 0.5
    pts = jnp.concatenate([pts, jnp.zeros((B, 1, 2), pts.dtype)], axis=1)
    px = pts[..., 0:1]
    py = pts[..., 1:2]
    labels = jnp.concatenate(
        [input_labels, -jnp.ones((B, 1), input_labels.dtype)],
        axis=1)[..., None]

    sparse, dense_hwc = pl.pallas_call(
        _body,
        in_specs=[
            pl.BlockSpec((B, NPTS, 1), lambda: (0, 0, 0)),
            pl.BlockSpec((B, NPTS, 1), lambda: (0, 0, 0)),
            pl.BlockSpec((B, NPTS, 1), lambda: (0, 0, 0)),
            pl.BlockSpec((2, HIDDEN // 2), lambda: (0, 0)),
            pl.BlockSpec((4, HIDDEN), lambda: (0, 0)),
            pl.BlockSpec((1, HIDDEN), lambda: (0, 0)),
            pl.BlockSpec((1, HIDDEN), lambda: (0, 0)),
        ],
        out_specs=[
            pl.BlockSpec((B, NPTS, HIDDEN), lambda: (0, 0, 0)),
            pl.BlockSpec(memory_space=pltpu.MemorySpace.HBM),
        ],
        out_shape=[
            jax.ShapeDtypeStruct((B, NPTS, HIDDEN), jnp.float32),
            jax.ShapeDtypeStruct((B, GRID, GRID, HIDDEN), jnp.float32),
        ],
        scratch_shapes=[
            pltpu.MemorySpace.VMEM((GRID, GRID, HIDDEN), jnp.float32),
            pltpu.SemaphoreType.DMA((NSEM,)),
        ],
    )(px, py, labels, positional_embedding, point_embed,
      not_a_point_embed, no_mask_embed)
    dense = dense_hwc.transpose(0, 3, 1, 2)
    return sparse, dense
